# initial kernel scaffold (unmeasured)
import jax
import jax.numpy as jnp
from jax import lax
from jax.experimental import pallas as pl
from jax.experimental.pallas import tpu as pltpu

N_DEV = 4
M_GLOBAL = 4096
M_BLK = M_GLOBAL // N_DEV
K_SH = 1024
N_TOT = 8192
N_CHUNKS = 8
WC = N_TOT // N_CHUNKS

F8_MAX = 448.0


def kernel(x, w_mat):
    def body(x_hbm, w_hbm, out_hbm,
             x_vmem, w_vmem, comm, amax_buf,
             copy_sem, send_sems, recv_sems, asend, arecv):
        d = lax.axis_index("i")
        left = lax.rem(d + N_DEV - 1, N_DEV)
        right = lax.rem(d + 1, N_DEV)

        barrier = pltpu.get_barrier_semaphore()
        for nbr in (left, right):
            pl.semaphore_signal(barrier, inc=1, device_id=(nbr,),
                                device_id_type=pl.DeviceIdType.MESH)
        pl.semaphore_wait(barrier, 2)

        cp = pltpu.make_async_copy(x_hbm, x_vmem, copy_sem)
        cp.start()
        cp.wait()

        amax = jnp.float32(0.0)

        for c in range(N_CHUNKS):
            col = pl.ds(c * WC, WC)
            cpw = pltpu.make_async_copy(w_hbm.at[:, col], w_vmem, copy_sem)
            cpw.start()
            cpw.wait()

            def partial_block(b):
                xb = x_vmem[pl.ds(b * M_BLK, M_BLK), :]
                return jnp.dot(xb, w_vmem[:, :],
                               preferred_element_type=jnp.float32)

            b0 = lax.rem(d + N_DEV - 1, N_DEV)
            comm[0] = partial_block(b0)

            for s in range(N_DEV - 1):
                send_slot = s % 2
                recv_slot = (s + 1) % 2
                rdma = pltpu.make_async_remote_copy(
                    src_ref=comm.at[send_slot],
                    dst_ref=comm.at[recv_slot],
                    send_sem=send_sems.at[send_slot],
                    recv_sem=recv_sems.at[recv_slot],
                    device_id=(right,),
                    device_id_type=pl.DeviceIdType.MESH,
                )
                rdma.start()
                rdma.wait()

                b = lax.rem(d + 2 * N_DEV - 2 - s, N_DEV)
                acc = comm[recv_slot] + partial_block(b)
                if s < N_DEV - 2:
                    comm[recv_slot] = acc
                else:
                    y = jnp.maximum(acc, 0.0)
                    amax = jnp.maximum(amax, jnp.max(y))
                    comm[recv_slot] = y
                    cpo = pltpu.make_async_copy(
                        comm.at[recv_slot], out_hbm.at[:, col], copy_sem)
                    cpo.start()
                    cpo.wait()

        amax_buf[0] = jnp.full((8, 128), amax, dtype=jnp.float32)
        sends = []
        for r in (1, 2, 3):
            peer = lax.rem(d + r, N_DEV)
            snd = pltpu.make_async_remote_copy(
                src_ref=amax_buf.at[0],
                dst_ref=amax_buf.at[N_DEV - r],
                send_sem=asend.at[r],
                recv_sem=arecv.at[N_DEV - r],
                device_id=(peer,),
                device_id_type=pl.DeviceIdType.MESH,
            )
            snd.start()
            sends.append(snd)
        for j, snd in zip((1, 2, 3), sends):
            rcv = pltpu.make_async_remote_copy(
                src_ref=amax_buf.at[0],
                dst_ref=amax_buf.at[j],
                send_sem=asend.at[j],
                recv_sem=arecv.at[j],
                device_id=(d,),
                device_id_type=pl.DeviceIdType.MESH,
            )
            rcv.wait_recv()
            snd.wait_send()

        g_amax = jnp.maximum(jnp.max(amax_buf[:, :, :]), jnp.float32(1e-30))
        scale = g_amax / F8_MAX
        inv_scale = F8_MAX / g_amax

        for c in range(N_CHUNKS):
            col = pl.ds(c * WC, WC)
            cpi = pltpu.make_async_copy(out_hbm.at[:, col], comm.at[0],
                                        copy_sem)
            cpi.start()
            cpi.wait()
            t = comm[0] * inv_scale
            q = t.astype(jnp.float8_e4m3fn).astype(jnp.float32)
            comm[0] = q * scale
            cpo = pltpu.make_async_copy(comm.at[0], out_hbm.at[:, col],
                                        copy_sem)
            cpo.start()
            cpo.wait()

    return pl.pallas_call(
        body,
        out_shape=jax.ShapeDtypeStruct((M_BLK, N_TOT), jnp.float32),
        in_specs=[
            pl.BlockSpec(memory_space=pltpu.ANY),
            pl.BlockSpec(memory_space=pltpu.ANY),
        ],
        out_specs=pl.BlockSpec(memory_space=pltpu.ANY),
        scratch_shapes=[
            pltpu.VMEM((M_GLOBAL, K_SH), jnp.float32),
            pltpu.VMEM((K_SH, WC), jnp.float32),
            pltpu.VMEM((2, M_BLK, WC), jnp.float32),
            pltpu.VMEM((N_DEV, 8, 128), jnp.float32),
            pltpu.SemaphoreType.DMA,
            pltpu.SemaphoreType.DMA((2,)),
            pltpu.SemaphoreType.DMA((2,)),
            pltpu.SemaphoreType.DMA((N_DEV,)),
            pltpu.SemaphoreType.DMA((N_DEV,)),
        ],
        compiler_params=pltpu.CompilerParams(collective_id=0),
    )(x, w_mat)


# baseline (device time: 1310692 ns/iter reference)
import jax
import jax.numpy as jnp
from jax import lax
from jax.experimental import pallas as pl
from jax.experimental.pallas import tpu as pltpu

N_DEV = 4
M_GLOBAL = 4096
M_BLK = M_GLOBAL // N_DEV
K_SH = 1024
N_TOT = 8192
N_CHUNKS = 8
WC = N_TOT // N_CHUNKS

F8_MAX = 448.0


def kernel(x, w_mat):
    def body(x_hbm, w_hbm, out_hbm,
             x_vmem, w_vmem, comm, amax_buf,
             copy_sem, send_sems, recv_sems, asend, arecv):
        d = lax.axis_index("i")
        left = lax.rem(d + N_DEV - 1, N_DEV)
        right = lax.rem(d + 1, N_DEV)

        barrier = pltpu.get_barrier_semaphore()
        for nbr in (left, right):
            pl.semaphore_signal(barrier, inc=1, device_id=(nbr,),
                                device_id_type=pl.DeviceIdType.MESH)
        pl.semaphore_wait(barrier, 2)

        cp = pltpu.make_async_copy(x_hbm, x_vmem, copy_sem)
        cp.start()
        cp.wait()

        amax = jnp.float32(0.0)

        for c in range(N_CHUNKS):
            col = pl.ds(c * WC, WC)
            cpw = pltpu.make_async_copy(w_hbm.at[:, col], w_vmem, copy_sem)
            cpw.start()
            cpw.wait()

            def partial_block(b):
                xb = x_vmem[pl.ds(b * M_BLK, M_BLK), :]
                return jnp.dot(xb, w_vmem[:, :],
                               preferred_element_type=jnp.float32)

            b0 = lax.rem(d + N_DEV - 1, N_DEV)
            comm[0] = partial_block(b0)

            for s in range(N_DEV - 1):
                send_slot = s % 2
                recv_slot = (s + 1) % 2
                rdma = pltpu.make_async_remote_copy(
                    src_ref=comm.at[send_slot],
                    dst_ref=comm.at[recv_slot],
                    send_sem=send_sems.at[send_slot],
                    recv_sem=recv_sems.at[recv_slot],
                    device_id=(right,),
                    device_id_type=pl.DeviceIdType.MESH,
                )
                rdma.start()
                rdma.wait()

                b = lax.rem(d + 2 * N_DEV - 2 - s, N_DEV)
                acc = comm[recv_slot] + partial_block(b)
                if s < N_DEV - 2:
                    comm[recv_slot] = acc
                else:
                    y = jnp.maximum(acc, 0.0)
                    amax = jnp.maximum(amax, jnp.max(y))
                    comm[recv_slot] = y
                    cpo = pltpu.make_async_copy(
                        comm.at[recv_slot], out_hbm.at[:, col], copy_sem)
                    cpo.start()
                    cpo.wait()

        amax_buf[0] = jnp.full((8, 128), amax, dtype=jnp.float32)
        sends = []
        for r in (1, 2, 3):
            peer = lax.rem(d + r, N_DEV)
            snd = pltpu.make_async_remote_copy(
                src_ref=amax_buf.at[0],
                dst_ref=amax_buf.at[N_DEV - r],
                send_sem=asend.at[r],
                recv_sem=arecv.at[N_DEV - r],
                device_id=(peer,),
                device_id_type=pl.DeviceIdType.MESH,
            )
            snd.start()
            sends.append(snd)
        for j, snd in zip((1, 2, 3), sends):
            rcv = pltpu.make_async_remote_copy(
                src_ref=amax_buf.at[0],
                dst_ref=amax_buf.at[j],
                send_sem=asend.at[j],
                recv_sem=arecv.at[j],
                device_id=(d,),
                device_id_type=pl.DeviceIdType.MESH,
            )
            rcv.wait_recv()
            snd.wait_send()

        g_amax = jnp.maximum(jnp.max(amax_buf[:, :, :]), jnp.float32(1e-30))
        scale = g_amax / F8_MAX
        inv_scale = F8_MAX / g_amax

        for c in range(N_CHUNKS):
            col = pl.ds(c * WC, WC)
            cpi = pltpu.make_async_copy(out_hbm.at[:, col], comm.at[0],
                                        copy_sem)
            cpi.start()
            cpi.wait()
            t = comm[0] * inv_scale
            q = t.astype(jnp.float8_e4m3fn).astype(jnp.float32)
            comm[0] = q * scale
            cpo = pltpu.make_async_copy(comm.at[0], out_hbm.at[:, col],
                                        copy_sem)
            cpo.start()
            cpo.wait()

    return pl.pallas_call(
        body,
        out_shape=jax.ShapeDtypeStruct((M_BLK, N_TOT), jnp.float32),
        in_specs=[
            pl.BlockSpec(memory_space=pl.ANY),
            pl.BlockSpec(memory_space=pl.ANY),
        ],
        out_specs=pl.BlockSpec(memory_space=pl.ANY),
        scratch_shapes=[
            pltpu.MemorySpace.VMEM((M_GLOBAL, K_SH), jnp.float32),
            pltpu.MemorySpace.VMEM((K_SH, WC), jnp.float32),
            pltpu.MemorySpace.VMEM((2, M_BLK, WC), jnp.float32),
            pltpu.MemorySpace.VMEM((N_DEV, 8, 128), jnp.float32),
            pltpu.SemaphoreType.DMA,
            pltpu.SemaphoreType.DMA((2,)),
            pltpu.SemaphoreType.DMA((2,)),
            pltpu.SemaphoreType.DMA((N_DEV,)),
            pltpu.SemaphoreType.DMA((N_DEV,)),
        ],
        compiler_params=pltpu.CompilerParams(collective_id=0),
    )(x, w_mat)
